# Initial kernel scaffold; baseline (speedup 1.0000x reference)
import jax
import jax.numpy as jnp
from jax.experimental import pallas as pl


def kernel(x, edge_index, W, b):
    raise NotImplementedError("write your pallas kernel here")


# speedup vs baseline: 16.0243x; 16.0243x over previous
import functools

import jax
import jax.numpy as jnp
from jax import lax
from jax.experimental import pallas as pl
from jax.experimental.pallas import tpu as pltpu
from jax.experimental.pallas import tpu_sc as plsc

N = 10000
D = 128
C = 40

NC = 2
NS = 16
NW = NC * NS
CHUNK = 128
PAD_ROWS = 112
NTOT = N + PAD_ROWS
DEG_W = 16

_SLAB = NTOT // NS


def _pad_edges(e):
    block = NW * CHUNK
    return ((e + block - 1) // block) * block


@functools.lru_cache(maxsize=None)
def _make_sc_degree(e_pad):
    per_w = e_pad // NW
    n_chunks = per_w // CHUNK
    mesh = plsc.VectorSubcoreMesh(core_axis_name="c", subcore_axis_name="s")

    @functools.partial(
        pl.kernel,
        mesh=mesh,
        out_type=jax.ShapeDtypeStruct((NC * NTOT, DEG_W), jnp.float32),
        compiler_params=pltpu.CompilerParams(use_tc_tiling_on_sc=False),
        scratch_types=[
            pltpu.VMEM((CHUNK,), jnp.int32),
            pltpu.VMEM((CHUNK, DEG_W), jnp.float32),
            pltpu.VMEM_SHARED((NTOT, DEG_W), jnp.float32),
        ],
    )
    def deg_kernel(didx_hbm, ones_hbm, zeros_hbm, out_hbm, didx_v, ones_v, acc):
        cid = lax.axis_index("c")
        sid = lax.axis_index("s")
        wid = sid * NC + cid
        pltpu.sync_copy(
            zeros_hbm.at[pl.ds(sid * _SLAB, _SLAB)],
            acc.at[pl.ds(sid * _SLAB, _SLAB)],
        )
        pltpu.sync_copy(ones_hbm, ones_v)
        plsc.subcore_barrier()
        wstart = wid * per_w

        def body(i, carry):
            off = wstart + i * CHUNK
            pltpu.sync_copy(didx_hbm.at[pl.ds(off, CHUNK)], didx_v)
            pltpu.sync_copy(ones_v, acc.at[didx_v], add=True)
            return carry

        lax.fori_loop(0, n_chunks, body, 0)
        plsc.subcore_barrier()
        pltpu.sync_copy(
            acc.at[pl.ds(sid * _SLAB, _SLAB)],
            out_hbm.at[pl.ds(cid * NTOT + sid * _SLAB, _SLAB)],
        )

    return deg_kernel


@functools.lru_cache(maxsize=None)
def _make_sc_prop(e_pad):
    per_w = e_pad // NW
    n_chunks = per_w // CHUNK
    mesh = plsc.VectorSubcoreMesh(core_axis_name="c", subcore_axis_name="s")

    @functools.partial(
        pl.kernel,
        mesh=mesh,
        out_type=jax.ShapeDtypeStruct((NC * NTOT, C), jnp.float32),
        compiler_params=pltpu.CompilerParams(use_tc_tiling_on_sc=False),
        scratch_types=[
            pltpu.VMEM((CHUNK,), jnp.int32),
            pltpu.VMEM((CHUNK,), jnp.int32),
            pltpu.VMEM((CHUNK, C), jnp.float32),
            pltpu.VMEM_SHARED((NTOT, C), jnp.float32),
            pltpu.SemaphoreType.DMA,
        ],
    )
    def prop_kernel(h_hbm, sidx_hbm, didx_hbm, zeros_hbm, out_hbm,
                    sidx_v, didx_v, rows_v, acc, sem):
        cid = lax.axis_index("c")
        sid = lax.axis_index("s")
        wid = sid * NC + cid
        pltpu.sync_copy(
            zeros_hbm.at[pl.ds(sid * _SLAB, _SLAB)],
            acc.at[pl.ds(sid * _SLAB, _SLAB)],
        )
        plsc.subcore_barrier()
        wstart = wid * per_w

        def body(i, carry):
            off = wstart + i * CHUNK
            pltpu.sync_copy(sidx_hbm.at[pl.ds(off, CHUNK)], sidx_v)
            pltpu.sync_copy(didx_hbm.at[pl.ds(off, CHUNK)], didx_v)
            pltpu.async_copy(h_hbm.at[sidx_v], rows_v, sem).wait()
            pltpu.sync_copy(rows_v, acc.at[didx_v], add=True)
            return carry

        lax.fori_loop(0, n_chunks, body, 0)
        plsc.subcore_barrier()
        pltpu.sync_copy(
            acc.at[pl.ds(sid * _SLAB, _SLAB)],
            out_hbm.at[pl.ds(cid * NTOT + sid * _SLAB, _SLAB)],
        )

    return prop_kernel


_BR = 400


def _tc_head(x, W, dp0, dp1):
    def body(x_ref, w_ref, d0_ref, d1_ref, z_ref, dinv_ref):
        deg = d0_ref[:, 0:1] + d1_ref[:, 0:1] + 1.0
        dinv = lax.rsqrt(deg)
        y = jnp.dot(x_ref[...], w_ref[...], preferred_element_type=jnp.float32)
        z_ref[...] = y * dinv
        dinv_ref[...] = jnp.broadcast_to(dinv, (_BR, 8))

    return pl.pallas_call(
        body,
        grid=(N // _BR,),
        in_specs=[
            pl.BlockSpec((_BR, D), lambda i: (i, 0)),
            pl.BlockSpec((D, C), lambda i: (0, 0)),
            pl.BlockSpec((_BR, DEG_W), lambda i: (i, 0)),
            pl.BlockSpec((_BR, DEG_W), lambda i: (i, 0)),
        ],
        out_specs=[
            pl.BlockSpec((_BR, C), lambda i: (i, 0)),
            pl.BlockSpec((_BR, 8), lambda i: (i, 0)),
        ],
        out_shape=[
            jax.ShapeDtypeStruct((N, C), jnp.float32),
            jax.ShapeDtypeStruct((N, 8), jnp.float32),
        ],
    )(x, W, dp0, dp1)


def _tc_mid(a0, a1, z0, dinv):
    def body(a0_ref, a1_ref, z_ref, dinv_ref, o_ref):
        d = dinv_ref[:, 0:1]
        o_ref[...] = (a0_ref[...] + a1_ref[...] + z_ref[...]) * (d * d)

    return pl.pallas_call(
        body,
        grid=(N // _BR,),
        in_specs=[
            pl.BlockSpec((_BR, C), lambda i: (i, 0)),
            pl.BlockSpec((_BR, C), lambda i: (i, 0)),
            pl.BlockSpec((_BR, C), lambda i: (i, 0)),
            pl.BlockSpec((_BR, 8), lambda i: (i, 0)),
        ],
        out_specs=pl.BlockSpec((_BR, C), lambda i: (i, 0)),
        out_shape=jax.ShapeDtypeStruct((N, C), jnp.float32),
    )(a0, a1, z0, dinv)


def _tc_final(a0, a1, z1, dinv, b2d):
    def body(a0_ref, a1_ref, z_ref, dinv_ref, b_ref, o_ref):
        d = dinv_ref[:, 0:1]
        t = (a0_ref[...] + a1_ref[...] + z_ref[...]) * d + b_ref[0:1, :]
        m = jnp.max(t, axis=1, keepdims=True)
        e = jnp.exp(t - m)
        s = jnp.sum(e, axis=1, keepdims=True)
        o_ref[...] = t - m - jnp.log(s)

    return pl.pallas_call(
        body,
        grid=(N // _BR,),
        in_specs=[
            pl.BlockSpec((_BR, C), lambda i: (i, 0)),
            pl.BlockSpec((_BR, C), lambda i: (i, 0)),
            pl.BlockSpec((_BR, C), lambda i: (i, 0)),
            pl.BlockSpec((_BR, 8), lambda i: (i, 0)),
            pl.BlockSpec((8, C), lambda i: (0, 0)),
        ],
        out_specs=pl.BlockSpec((_BR, C), lambda i: (i, 0)),
        out_shape=jax.ShapeDtypeStruct((N, C), jnp.float32),
    )(a0, a1, z1, dinv, b2d)


def kernel(x, edge_index, W, b):
    src = edge_index[0]
    dst = edge_index[1]
    e = src.shape[0]
    e_pad = _pad_edges(e)
    pad = e_pad - e
    src_p = jnp.concatenate([src, jnp.zeros((pad,), jnp.int32)])
    dst_p = jnp.concatenate(
        [dst, N + (jnp.arange(pad, dtype=jnp.int32) % PAD_ROWS)])
    ones_rows = jnp.ones((CHUNK, DEG_W), jnp.float32)
    zeros_deg = jnp.zeros((NTOT, DEG_W), jnp.float32)
    zeros_c = jnp.zeros((NTOT, C), jnp.float32)
    b2d = jnp.broadcast_to(b[None, :], (8, C))

    degp = _make_sc_degree(e_pad)(dst_p, ones_rows, zeros_deg)
    z0, dinv = _tc_head(x, W, degp[:N], degp[NTOT:NTOT + N])
    prop = _make_sc_prop(e_pad)
    a1 = prop(z0, src_p, dst_p, zeros_c)
    z1 = _tc_mid(a1[:N], a1[NTOT:NTOT + N], z0, dinv)
    a2 = prop(z1, src_p, dst_p, zeros_c)
    return _tc_final(a2[:N], a2[NTOT:NTOT + N], z1, dinv, b2d)


# speedup vs baseline: 34.3263x; 2.1421x over previous
import functools

import jax
import jax.numpy as jnp
from jax import lax
from jax.experimental import pallas as pl
from jax.experimental.pallas import tpu as pltpu
from jax.experimental.pallas import tpu_sc as plsc

N = 10000
D = 128
C = 40

NC = 2
NS = 16
NW = NC * NS
CHUNK = 128
NBUF = 4
PAD_ROWS = 112
NTOT = N + PAD_ROWS
DEG_W = 16

_SLAB = NTOT // NS
_HSTAGE = 624


def _pad_edges(e):
    block = NW * CHUNK * NBUF
    return ((e + block - 1) // block) * block


def _zero_rows(zbuf, n_rows, width):
    zv = jnp.zeros((16,), jnp.float32)
    cols = [0] if width == 16 else [0, 16, width - 16]

    def body(r, carry):
        for c in cols:
            zbuf[r, pl.ds(c, 16)] = zv
        return carry

    lax.fori_loop(0, n_rows, body, 0)


@functools.lru_cache(maxsize=None)
def _make_sc_degree(e_pad):
    per_w = e_pad // NW
    n_chunks = per_w // CHUNK
    lag = 8
    mesh = plsc.VectorSubcoreMesh(core_axis_name="c", subcore_axis_name="s")

    @functools.partial(
        pl.kernel,
        mesh=mesh,
        out_type=jax.ShapeDtypeStruct((NC * NTOT, DEG_W), jnp.float32),
        compiler_params=pltpu.CompilerParams(use_tc_tiling_on_sc=False),
        scratch_types=[
            pltpu.VMEM((n_chunks, CHUNK), jnp.int32),
            pltpu.VMEM((CHUNK, DEG_W), jnp.float32),
            pltpu.VMEM((CHUNK, DEG_W), jnp.float32),
            pltpu.VMEM_SHARED((NTOT, DEG_W), jnp.float32),
            pltpu.SemaphoreType.DMA,
        ],
    )
    def deg_kernel(didx_hbm, ones_hbm, out_hbm, didx_all, ones_v, zbuf, acc,
                   sem):
        cid = lax.axis_index("c")
        sid = lax.axis_index("s")
        wid = sid * NC + cid
        _zero_rows(zbuf, CHUNK, DEG_W)
        base = sid * _SLAB
        for j in range(4):
            pltpu.sync_copy(zbuf, acc.at[pl.ds(base + j * CHUNK, CHUNK)])
        pltpu.sync_copy(zbuf.at[pl.ds(0, _SLAB - 4 * CHUNK)],
                        acc.at[pl.ds(base + 4 * CHUNK, _SLAB - 4 * CHUNK)])
        pltpu.sync_copy(ones_hbm, ones_v)
        pltpu.sync_copy(didx_hbm.at[wid], didx_all)
        plsc.subcore_barrier()
        for i in range(n_chunks):
            pltpu.async_copy(ones_v, acc.at[didx_all.at[i]], sem, add=True)
            if i >= lag:
                pltpu.make_async_copy(
                    ones_v, acc.at[didx_all.at[i - lag]], sem).wait()
        for i in range(n_chunks - lag, n_chunks):
            pltpu.make_async_copy(ones_v, acc.at[didx_all.at[i]], sem).wait()
        plsc.subcore_barrier()
        pltpu.sync_copy(
            acc.at[pl.ds(sid * _SLAB, _SLAB)],
            out_hbm.at[pl.ds(cid * NTOT + sid * _SLAB, _SLAB)],
        )

    return deg_kernel


@functools.lru_cache(maxsize=None)
def _make_sc_prop(e_pad):
    per_w = e_pad // NW
    n_chunks = per_w // CHUNK
    n_groups = n_chunks // NBUF
    mesh = plsc.VectorSubcoreMesh(core_axis_name="c", subcore_axis_name="s")

    @functools.partial(
        pl.kernel,
        mesh=mesh,
        out_type=jax.ShapeDtypeStruct((NC * NTOT, C), jnp.float32),
        compiler_params=pltpu.CompilerParams(use_tc_tiling_on_sc=False),
        scratch_types=[
            pltpu.VMEM((n_chunks, CHUNK), jnp.int32),
            pltpu.VMEM((n_chunks, CHUNK), jnp.int32),
            pltpu.VMEM((NBUF, CHUNK, C), jnp.float32),
            pltpu.VMEM((CHUNK, C), jnp.float32),
            pltpu.VMEM_SHARED((N, C), jnp.float32),
            pltpu.VMEM_SHARED((NTOT, C), jnp.float32),
            pltpu.SemaphoreType.DMA((NBUF,)),
            pltpu.SemaphoreType.DMA((NBUF,)),
        ],
    )
    def prop_kernel(h_hbm, sidx_hbm, didx_hbm, out_hbm,
                    sidx_all, didx_all, rows, zbuf, h_sp, acc, gsem, ssem):
        cid = lax.axis_index("c")
        sid = lax.axis_index("s")
        wid = sid * NC + cid
        pltpu.sync_copy(h_hbm.at[pl.ds(sid * _HSTAGE, _HSTAGE)],
                        h_sp.at[pl.ds(sid * _HSTAGE, _HSTAGE)])

        @pl.when(sid == NS - 1)
        def _():
            pltpu.sync_copy(h_hbm.at[pl.ds(NS * _HSTAGE, N - NS * _HSTAGE)],
                            h_sp.at[pl.ds(NS * _HSTAGE, N - NS * _HSTAGE)])

        _zero_rows(zbuf, CHUNK, C)
        base = sid * _SLAB
        for j in range(4):
            pltpu.sync_copy(zbuf, acc.at[pl.ds(base + j * CHUNK, CHUNK)])
        pltpu.sync_copy(zbuf.at[pl.ds(0, _SLAB - 4 * CHUNK)],
                        acc.at[pl.ds(base + 4 * CHUNK, _SLAB - 4 * CHUNK)])
        pltpu.sync_copy(sidx_hbm.at[wid], sidx_all)
        pltpu.sync_copy(didx_hbm.at[wid], didx_all)
        plsc.subcore_barrier()

        def gather(i, b):
            pltpu.async_copy(h_sp.at[sidx_all.at[i]], rows.at[b], gsem.at[b])

        def gather_wait(i, b):
            pltpu.make_async_copy(
                h_sp.at[sidx_all.at[i]], rows.at[b], gsem.at[b]).wait()

        def scatter(i, b):
            pltpu.async_copy(rows.at[b], acc.at[didx_all.at[i]], ssem.at[b],
                             add=True)

        def scatter_wait(i, b):
            pltpu.make_async_copy(
                rows.at[b], acc.at[didx_all.at[i]], ssem.at[b]).wait()

        for b in range(NBUF):
            gather(b, b)

        def body(g, carry):
            i0 = g * NBUF
            for b in range(NBUF):
                gather_wait(i0 + b, b)
                scatter(i0 + b, b)
            for b in range(NBUF):
                scatter_wait(i0 + b, b)
                gather(i0 + NBUF + b, b)
            return carry

        lax.fori_loop(0, n_groups - 1, body, 0)
        i0 = (n_groups - 1) * NBUF
        for b in range(NBUF):
            gather_wait(i0 + b, b)
            scatter(i0 + b, b)
        for b in range(NBUF):
            scatter_wait(i0 + b, b)
        plsc.subcore_barrier()
        pltpu.sync_copy(
            acc.at[pl.ds(sid * _SLAB, _SLAB)],
            out_hbm.at[pl.ds(cid * NTOT + sid * _SLAB, _SLAB)],
        )

    return prop_kernel


_BR = 400


def _tc_head(x, W, dp0, dp1):
    def body(x_ref, w_ref, d0_ref, d1_ref, z_ref, dinv_ref):
        deg = d0_ref[:, 0:1] + d1_ref[:, 0:1] + 1.0
        dinv = lax.rsqrt(deg)
        y = jnp.dot(x_ref[...], w_ref[...], preferred_element_type=jnp.float32)
        z_ref[...] = y * dinv
        dinv_ref[...] = jnp.broadcast_to(dinv, (_BR, 8))

    return pl.pallas_call(
        body,
        grid=(N // _BR,),
        in_specs=[
            pl.BlockSpec((_BR, D), lambda i: (i, 0)),
            pl.BlockSpec((D, C), lambda i: (0, 0)),
            pl.BlockSpec((_BR, DEG_W), lambda i: (i, 0)),
            pl.BlockSpec((_BR, DEG_W), lambda i: (i, 0)),
        ],
        out_specs=[
            pl.BlockSpec((_BR, C), lambda i: (i, 0)),
            pl.BlockSpec((_BR, 8), lambda i: (i, 0)),
        ],
        out_shape=[
            jax.ShapeDtypeStruct((N, C), jnp.float32),
            jax.ShapeDtypeStruct((N, 8), jnp.float32),
        ],
    )(x, W, dp0, dp1)


def _tc_mid(a0, a1, z0, dinv):
    def body(a0_ref, a1_ref, z_ref, dinv_ref, o_ref):
        d = dinv_ref[:, 0:1]
        o_ref[...] = (a0_ref[...] + a1_ref[...] + z_ref[...]) * (d * d)

    return pl.pallas_call(
        body,
        grid=(N // _BR,),
        in_specs=[
            pl.BlockSpec((_BR, C), lambda i: (i, 0)),
            pl.BlockSpec((_BR, C), lambda i: (i, 0)),
            pl.BlockSpec((_BR, C), lambda i: (i, 0)),
            pl.BlockSpec((_BR, 8), lambda i: (i, 0)),
        ],
        out_specs=pl.BlockSpec((_BR, C), lambda i: (i, 0)),
        out_shape=jax.ShapeDtypeStruct((N, C), jnp.float32),
    )(a0, a1, z0, dinv)


def _tc_final(a0, a1, z1, dinv, b2d):
    def body(a0_ref, a1_ref, z_ref, dinv_ref, b_ref, o_ref):
        d = dinv_ref[:, 0:1]
        t = (a0_ref[...] + a1_ref[...] + z_ref[...]) * d + b_ref[0:1, :]
        m = jnp.max(t, axis=1, keepdims=True)
        e = jnp.exp(t - m)
        s = jnp.sum(e, axis=1, keepdims=True)
        o_ref[...] = t - m - jnp.log(s)

    return pl.pallas_call(
        body,
        grid=(N // _BR,),
        in_specs=[
            pl.BlockSpec((_BR, C), lambda i: (i, 0)),
            pl.BlockSpec((_BR, C), lambda i: (i, 0)),
            pl.BlockSpec((_BR, C), lambda i: (i, 0)),
            pl.BlockSpec((_BR, 8), lambda i: (i, 0)),
            pl.BlockSpec((8, C), lambda i: (0, 0)),
        ],
        out_specs=pl.BlockSpec((_BR, C), lambda i: (i, 0)),
        out_shape=jax.ShapeDtypeStruct((N, C), jnp.float32),
    )(a0, a1, z1, dinv, b2d)


def kernel(x, edge_index, W, b):
    src = edge_index[0]
    dst = edge_index[1]
    e = src.shape[0]
    e_pad = _pad_edges(e)
    pad = e_pad - e
    per_w = e_pad // NW
    n_chunks = per_w // CHUNK
    pad_i = jnp.arange(pad, dtype=jnp.int32)
    src_p = jnp.concatenate([src, (pad_i * 37) % N]).reshape(
        NW, n_chunks, CHUNK)
    dst_p = jnp.concatenate([dst, N + pad_i % PAD_ROWS]).reshape(
        NW, n_chunks, CHUNK)
    ones_rows = jnp.ones((CHUNK, DEG_W), jnp.float32)
    b2d = jnp.broadcast_to(b[None, :], (8, C))

    degp = _make_sc_degree(e_pad)(dst_p, ones_rows)
    z0, dinv = _tc_head(x, W, degp[:N], degp[NTOT:NTOT + N])
    prop = _make_sc_prop(e_pad)
    a1 = prop(z0, src_p, dst_p)
    z1 = _tc_mid(a1[:N], a1[NTOT:NTOT + N], z0, dinv)
    a2 = prop(z1, src_p, dst_p)
    return _tc_final(a2[:N], a2[NTOT:NTOT + N], z1, dinv, b2d)


# speedup vs baseline: 38.8450x; 1.1316x over previous
import functools

import jax
import jax.numpy as jnp
from jax import lax
from jax.experimental import pallas as pl
from jax.experimental.pallas import tpu as pltpu
from jax.experimental.pallas import tpu_sc as plsc

N = 10000
D = 128
C = 40

NC = 2
NS = 16
NW = NC * NS
CHUNK = 128
NBUF = 8
PAD_ROWS = 112
NTOT = N + PAD_ROWS
DEG_W = 16

_SLAB = NTOT // NS
_HSTAGE = 624


def _pad_edges(e):
    block = NW * CHUNK * NBUF
    return ((e + block - 1) // block) * block


def _zero_rows(zbuf, n_rows, width):
    zv = jnp.zeros((16,), jnp.float32)
    cols = [0] if width == 16 else [0, 16, width - 16]

    def body(r, carry):
        for c in cols:
            zbuf[r, pl.ds(c, 16)] = zv
        return carry

    lax.fori_loop(0, n_rows, body, 0)


@functools.lru_cache(maxsize=None)
def _make_sc_degree(e_pad):
    per_w = e_pad // NW
    n_chunks = per_w // CHUNK
    lag = 8
    mesh = plsc.VectorSubcoreMesh(core_axis_name="c", subcore_axis_name="s")

    @functools.partial(
        pl.kernel,
        mesh=mesh,
        out_type=jax.ShapeDtypeStruct((NC * NTOT, DEG_W), jnp.float32),
        compiler_params=pltpu.CompilerParams(use_tc_tiling_on_sc=False),
        scratch_types=[
            pltpu.VMEM((n_chunks, CHUNK), jnp.int32),
            pltpu.VMEM((CHUNK, DEG_W), jnp.float32),
            pltpu.VMEM((CHUNK, DEG_W), jnp.float32),
            pltpu.VMEM_SHARED((NTOT, DEG_W), jnp.float32),
            pltpu.SemaphoreType.DMA,
        ],
    )
    def deg_kernel(didx_hbm, ones_hbm, out_hbm, didx_all, ones_v, zbuf, acc,
                   sem):
        cid = lax.axis_index("c")
        sid = lax.axis_index("s")
        wid = sid * NC + cid
        _zero_rows(zbuf, CHUNK, DEG_W)
        base = sid * _SLAB
        for j in range(4):
            pltpu.sync_copy(zbuf, acc.at[pl.ds(base + j * CHUNK, CHUNK)])
        pltpu.sync_copy(zbuf.at[pl.ds(0, _SLAB - 4 * CHUNK)],
                        acc.at[pl.ds(base + 4 * CHUNK, _SLAB - 4 * CHUNK)])
        pltpu.sync_copy(ones_hbm, ones_v)
        pltpu.sync_copy(didx_hbm.at[wid], didx_all)
        plsc.subcore_barrier()
        for i in range(n_chunks):
            pltpu.async_copy(ones_v, acc.at[didx_all.at[i]], sem, add=True)
            if i >= lag:
                pltpu.make_async_copy(
                    ones_v, acc.at[didx_all.at[i - lag]], sem).wait()
        for i in range(n_chunks - lag, n_chunks):
            pltpu.make_async_copy(ones_v, acc.at[didx_all.at[i]], sem).wait()
        plsc.subcore_barrier()
        pltpu.sync_copy(
            acc.at[pl.ds(sid * _SLAB, _SLAB)],
            out_hbm.at[pl.ds(cid * NTOT + sid * _SLAB, _SLAB)],
        )

    return deg_kernel


@functools.lru_cache(maxsize=None)
def _make_sc_prop(e_pad):
    per_w = e_pad // NW
    n_chunks = per_w // CHUNK
    n_groups = n_chunks // NBUF
    mesh = plsc.VectorSubcoreMesh(core_axis_name="c", subcore_axis_name="s")

    @functools.partial(
        pl.kernel,
        mesh=mesh,
        out_type=jax.ShapeDtypeStruct((NC * NTOT, C), jnp.float32),
        compiler_params=pltpu.CompilerParams(use_tc_tiling_on_sc=False),
        scratch_types=[
            pltpu.VMEM((n_chunks, CHUNK), jnp.int32),
            pltpu.VMEM((n_chunks, CHUNK), jnp.int32),
            pltpu.VMEM((NBUF, CHUNK, C), jnp.float32),
            pltpu.VMEM((CHUNK, C), jnp.float32),
            pltpu.VMEM_SHARED((N, C), jnp.float32),
            pltpu.VMEM_SHARED((NTOT, C), jnp.float32),
            pltpu.SemaphoreType.DMA((NBUF,)),
            pltpu.SemaphoreType.DMA((NBUF,)),
        ],
    )
    def prop_kernel(h_hbm, sidx_hbm, didx_hbm, out_hbm,
                    sidx_all, didx_all, rows, zbuf, h_sp, acc, gsem, ssem):
        cid = lax.axis_index("c")
        sid = lax.axis_index("s")
        wid = sid * NC + cid
        pltpu.sync_copy(h_hbm.at[pl.ds(sid * _HSTAGE, _HSTAGE)],
                        h_sp.at[pl.ds(sid * _HSTAGE, _HSTAGE)])

        @pl.when(sid == NS - 1)
        def _():
            pltpu.sync_copy(h_hbm.at[pl.ds(NS * _HSTAGE, N - NS * _HSTAGE)],
                            h_sp.at[pl.ds(NS * _HSTAGE, N - NS * _HSTAGE)])

        _zero_rows(zbuf, CHUNK, C)
        base = sid * _SLAB
        for j in range(4):
            pltpu.sync_copy(zbuf, acc.at[pl.ds(base + j * CHUNK, CHUNK)])
        pltpu.sync_copy(zbuf.at[pl.ds(0, _SLAB - 4 * CHUNK)],
                        acc.at[pl.ds(base + 4 * CHUNK, _SLAB - 4 * CHUNK)])
        pltpu.sync_copy(sidx_hbm.at[wid], sidx_all)
        pltpu.sync_copy(didx_hbm.at[wid], didx_all)
        plsc.subcore_barrier()

        def gather(i, b):
            pltpu.async_copy(h_sp.at[sidx_all.at[i]], rows.at[b], gsem.at[b])

        def gather_wait(i, b):
            pltpu.make_async_copy(
                h_sp.at[sidx_all.at[i]], rows.at[b], gsem.at[b]).wait()

        def scatter(i, b):
            pltpu.async_copy(rows.at[b], acc.at[didx_all.at[i]], ssem.at[b],
                             add=True)

        def scatter_wait(i, b):
            pltpu.make_async_copy(
                rows.at[b], acc.at[didx_all.at[i]], ssem.at[b]).wait()

        for b in range(NBUF):
            gather(b, b)

        def body(g, carry):
            i0 = g * NBUF
            for b in range(NBUF):
                gather_wait(i0 + b, b)
                scatter(i0 + b, b)
            for b in range(NBUF):
                scatter_wait(i0 + b, b)
                gather(i0 + NBUF + b, b)
            return carry

        lax.fori_loop(0, n_groups - 1, body, 0)
        i0 = (n_groups - 1) * NBUF
        for b in range(NBUF):
            gather_wait(i0 + b, b)
            scatter(i0 + b, b)
        for b in range(NBUF):
            scatter_wait(i0 + b, b)
        plsc.subcore_barrier()
        pltpu.sync_copy(
            acc.at[pl.ds(sid * _SLAB, _SLAB)],
            out_hbm.at[pl.ds(cid * NTOT + sid * _SLAB, _SLAB)],
        )

    return prop_kernel


_BR = 400


def _tc_head(x, W, degp):
    def body(x_ref, w_ref, d0_ref, d1_ref, z_ref, dinv_ref):
        deg = d0_ref[0, :, 0:1] + d1_ref[0, :, 0:1] + 1.0
        dinv = lax.rsqrt(deg)
        y = jnp.dot(x_ref[...], w_ref[...], preferred_element_type=jnp.float32)
        z_ref[...] = y * dinv
        dinv_ref[...] = jnp.broadcast_to(dinv, (_BR, 8))

    return pl.pallas_call(
        body,
        grid=(N // _BR,),
        in_specs=[
            pl.BlockSpec((_BR, D), lambda i: (i, 0)),
            pl.BlockSpec((D, C), lambda i: (0, 0)),
            pl.BlockSpec((1, _BR, DEG_W), lambda i: (0, i, 0)),
            pl.BlockSpec((1, _BR, DEG_W), lambda i: (1, i, 0)),
        ],
        out_specs=[
            pl.BlockSpec((_BR, C), lambda i: (i, 0)),
            pl.BlockSpec((_BR, 8), lambda i: (i, 0)),
        ],
        out_shape=[
            jax.ShapeDtypeStruct((N, C), jnp.float32),
            jax.ShapeDtypeStruct((N, 8), jnp.float32),
        ],
    )(x, W, degp, degp)


def _tc_mid(ap, z0, dinv):
    def body(a0_ref, a1_ref, z_ref, dinv_ref, o_ref):
        d = dinv_ref[:, 0:1]
        o_ref[...] = (a0_ref[0] + a1_ref[0] + z_ref[...]) * (d * d)

    return pl.pallas_call(
        body,
        grid=(N // _BR,),
        in_specs=[
            pl.BlockSpec((1, _BR, C), lambda i: (0, i, 0)),
            pl.BlockSpec((1, _BR, C), lambda i: (1, i, 0)),
            pl.BlockSpec((_BR, C), lambda i: (i, 0)),
            pl.BlockSpec((_BR, 8), lambda i: (i, 0)),
        ],
        out_specs=pl.BlockSpec((_BR, C), lambda i: (i, 0)),
        out_shape=jax.ShapeDtypeStruct((N, C), jnp.float32),
    )(ap, ap, z0, dinv)


def _tc_final(ap, z1, dinv, b2d):
    def body(a0_ref, a1_ref, z_ref, dinv_ref, b_ref, o_ref):
        d = dinv_ref[:, 0:1]
        t = (a0_ref[0] + a1_ref[0] + z_ref[...]) * d + b_ref[0:1, :]
        m = jnp.max(t, axis=1, keepdims=True)
        e = jnp.exp(t - m)
        s = jnp.sum(e, axis=1, keepdims=True)
        o_ref[...] = t - m - jnp.log(s)

    return pl.pallas_call(
        body,
        grid=(N // _BR,),
        in_specs=[
            pl.BlockSpec((1, _BR, C), lambda i: (0, i, 0)),
            pl.BlockSpec((1, _BR, C), lambda i: (1, i, 0)),
            pl.BlockSpec((_BR, C), lambda i: (i, 0)),
            pl.BlockSpec((_BR, 8), lambda i: (i, 0)),
            pl.BlockSpec((8, C), lambda i: (0, 0)),
        ],
        out_specs=pl.BlockSpec((_BR, C), lambda i: (i, 0)),
        out_shape=jax.ShapeDtypeStruct((N, C), jnp.float32),
    )(ap, ap, z1, dinv, b2d)


def kernel(x, edge_index, W, b):
    src = edge_index[0]
    dst = edge_index[1]
    e = src.shape[0]
    e_pad = _pad_edges(e)
    pad = e_pad - e
    per_w = e_pad // NW
    n_chunks = per_w // CHUNK
    pad_i = jnp.arange(pad, dtype=jnp.int32)
    src_p = jnp.concatenate([src, (pad_i * 37) % N]).reshape(
        NW, n_chunks, CHUNK)
    dst_p = jnp.concatenate([dst, N + pad_i % PAD_ROWS]).reshape(
        NW, n_chunks, CHUNK)
    ones_rows = jnp.ones((CHUNK, DEG_W), jnp.float32)
    b2d = jnp.broadcast_to(b[None, :], (8, C))

    degp = _make_sc_degree(e_pad)(dst_p, ones_rows).reshape(2, NTOT, DEG_W)
    z0, dinv = _tc_head(x, W, degp)
    prop = _make_sc_prop(e_pad)
    a1 = prop(z0, src_p, dst_p).reshape(2, NTOT, C)
    z1 = _tc_mid(a1, z0, dinv)
    a2 = prop(z1, src_p, dst_p).reshape(2, NTOT, C)
    return _tc_final(a2, z1, dinv, b2d)


# speedup vs baseline: 43.5271x; 1.1205x over previous
import functools

import jax
import jax.numpy as jnp
from jax import lax
from jax.experimental import pallas as pl
from jax.experimental.pallas import tpu as pltpu
from jax.experimental.pallas import tpu_sc as plsc

N = 10000
D = 128
C = 40

NC = 2
NS = 16
NW = NC * NS
CHUNK = 128
NBUF = 8
PAD_ROWS = 112
NTOT = N + PAD_ROWS
DEG_W = 16

_SLAB = NTOT // NS
_HSTAGE = 624


def _pad_edges(e):
    block = NW * CHUNK * NBUF
    return ((e + block - 1) // block) * block


def _zero_rows(zbuf, n_rows, width):
    zv = jnp.zeros((16,), jnp.float32)
    cols = [0] if width == 16 else [0, 16, width - 16]

    def body(r, carry):
        for c in cols:
            zbuf[r, pl.ds(c, 16)] = zv
        return carry

    lax.fori_loop(0, n_rows, body, 0)


@functools.lru_cache(maxsize=None)
def _make_sc_degree(e_pad):
    per_w = e_pad // NW
    n_chunks = per_w // CHUNK
    lag = 8
    mesh = plsc.VectorSubcoreMesh(core_axis_name="c", subcore_axis_name="s")

    @functools.partial(
        pl.kernel,
        mesh=mesh,
        out_type=jax.ShapeDtypeStruct((NC * NTOT, DEG_W), jnp.float32),
        compiler_params=pltpu.CompilerParams(use_tc_tiling_on_sc=False),
        scratch_types=[
            pltpu.VMEM((n_chunks, CHUNK), jnp.int32),
            pltpu.VMEM((CHUNK, DEG_W), jnp.float32),
            pltpu.VMEM((CHUNK, DEG_W), jnp.float32),
            pltpu.VMEM_SHARED((NTOT, DEG_W), jnp.float32),
            pltpu.SemaphoreType.DMA,
        ],
    )
    def deg_kernel(didx_hbm, ones_hbm, out_hbm, didx_all, ones_v, zbuf, acc,
                   sem):
        cid = lax.axis_index("c")
        sid = lax.axis_index("s")
        wid = sid * NC + cid
        _zero_rows(zbuf, CHUNK, DEG_W)
        base = sid * _SLAB
        for j in range(4):
            pltpu.sync_copy(zbuf, acc.at[pl.ds(base + j * CHUNK, CHUNK)])
        pltpu.sync_copy(zbuf.at[pl.ds(0, _SLAB - 4 * CHUNK)],
                        acc.at[pl.ds(base + 4 * CHUNK, _SLAB - 4 * CHUNK)])
        pltpu.sync_copy(ones_hbm, ones_v)
        pltpu.sync_copy(didx_hbm.at[wid], didx_all)
        plsc.subcore_barrier()
        for i in range(n_chunks):
            pltpu.async_copy(ones_v, acc.at[didx_all.at[i]], sem, add=True)
            if i >= lag:
                pltpu.make_async_copy(
                    ones_v, acc.at[didx_all.at[i - lag]], sem).wait()
        for i in range(n_chunks - lag, n_chunks):
            pltpu.make_async_copy(ones_v, acc.at[didx_all.at[i]], sem).wait()
        plsc.subcore_barrier()
        pltpu.sync_copy(
            acc.at[pl.ds(sid * _SLAB, _SLAB)],
            out_hbm.at[pl.ds(cid * NTOT + sid * _SLAB, _SLAB)],
        )

    return deg_kernel


@functools.lru_cache(maxsize=None)
def _make_sc_prop(e_pad):
    per_w = e_pad // NW
    n_chunks = per_w // CHUNK
    n_groups = n_chunks // NBUF
    mesh = plsc.VectorSubcoreMesh(core_axis_name="c", subcore_axis_name="s")

    @functools.partial(
        pl.kernel,
        mesh=mesh,
        out_type=jax.ShapeDtypeStruct((NC * NTOT, C), jnp.float32),
        compiler_params=pltpu.CompilerParams(use_tc_tiling_on_sc=False),
        scratch_types=[
            pltpu.VMEM((n_chunks, CHUNK), jnp.int32),
            pltpu.VMEM((n_chunks, CHUNK), jnp.int32),
            pltpu.VMEM((NBUF, CHUNK, C), jnp.float32),
            pltpu.VMEM((CHUNK, C), jnp.float32),
            pltpu.VMEM_SHARED((N, C), jnp.float32),
            pltpu.VMEM_SHARED((NTOT, C), jnp.float32),
            pltpu.SemaphoreType.DMA((NBUF,)),
            pltpu.SemaphoreType.DMA((NBUF,)),
        ],
    )
    def prop_kernel(h_hbm, sidx_hbm, didx_hbm, out_hbm,
                    sidx_all, didx_all, rows, zbuf, h_sp, acc, gsem, ssem):
        cid = lax.axis_index("c")
        sid = lax.axis_index("s")
        wid = sid * NC + cid
        pltpu.sync_copy(h_hbm.at[pl.ds(sid * _HSTAGE, _HSTAGE)],
                        h_sp.at[pl.ds(sid * _HSTAGE, _HSTAGE)])

        @pl.when(sid == NS - 1)
        def _():
            pltpu.sync_copy(h_hbm.at[pl.ds(NS * _HSTAGE, N - NS * _HSTAGE)],
                            h_sp.at[pl.ds(NS * _HSTAGE, N - NS * _HSTAGE)])

        _zero_rows(zbuf, CHUNK, C)
        base = sid * _SLAB
        for j in range(4):
            pltpu.sync_copy(zbuf, acc.at[pl.ds(base + j * CHUNK, CHUNK)])
        pltpu.sync_copy(zbuf.at[pl.ds(0, _SLAB - 4 * CHUNK)],
                        acc.at[pl.ds(base + 4 * CHUNK, _SLAB - 4 * CHUNK)])
        pltpu.sync_copy(sidx_hbm.at[wid], sidx_all)
        pltpu.sync_copy(didx_hbm.at[wid], didx_all)
        plsc.subcore_barrier()

        def gather(i, b):
            pltpu.async_copy(h_sp.at[sidx_all.at[i]], rows.at[b], gsem.at[b])

        def gather_wait(i, b):
            pltpu.make_async_copy(
                h_sp.at[sidx_all.at[i]], rows.at[b], gsem.at[b]).wait()

        def scatter(i, b):
            pltpu.async_copy(rows.at[b], acc.at[didx_all.at[i]], ssem.at[b],
                             add=True)

        def scatter_wait(i, b):
            pltpu.make_async_copy(
                rows.at[b], acc.at[didx_all.at[i]], ssem.at[b]).wait()

        for b in range(NBUF):
            gather(b, b)

        def body(g, carry):
            i0 = g * NBUF
            for b in range(NBUF):
                gather_wait(i0 + b, b)
                scatter(i0 + b, b)
            for b in range(NBUF):
                scatter_wait(i0 + b, b)
                gather(i0 + NBUF + b, b)
            return carry

        lax.fori_loop(0, n_groups - 1, body, 0)
        i0 = (n_groups - 1) * NBUF
        for b in range(NBUF):
            gather_wait(i0 + b, b)
            scatter(i0 + b, b)
        for b in range(NBUF):
            scatter_wait(i0 + b, b)
        plsc.subcore_barrier()
        pltpu.sync_copy(
            acc.at[pl.ds(sid * _SLAB, _SLAB)],
            out_hbm.at[pl.ds(cid * NTOT + sid * _SLAB, _SLAB)],
        )

    return prop_kernel


_BR = 2000


def _tc_head(x, W, degp):
    def body(x_ref, w_ref, d0_ref, d1_ref, z_ref, dinv_ref):
        deg = d0_ref[0, :, 0:1] + d1_ref[0, :, 0:1] + 1.0
        dinv = lax.rsqrt(deg)
        y = jnp.dot(x_ref[...], w_ref[...], preferred_element_type=jnp.float32)
        z_ref[...] = y * dinv
        dinv_ref[...] = jnp.broadcast_to(dinv, (_BR, 8))

    return pl.pallas_call(
        body,
        grid=(N // _BR,),
        in_specs=[
            pl.BlockSpec((_BR, D), lambda i: (i, 0)),
            pl.BlockSpec((D, C), lambda i: (0, 0)),
            pl.BlockSpec((1, _BR, DEG_W), lambda i: (0, i, 0)),
            pl.BlockSpec((1, _BR, DEG_W), lambda i: (1, i, 0)),
        ],
        out_specs=[
            pl.BlockSpec((_BR, C), lambda i: (i, 0)),
            pl.BlockSpec((_BR, 8), lambda i: (i, 0)),
        ],
        out_shape=[
            jax.ShapeDtypeStruct((N, C), jnp.float32),
            jax.ShapeDtypeStruct((N, 8), jnp.float32),
        ],
    )(x, W, degp, degp)


def _tc_mid(ap, z0, dinv):
    def body(a0_ref, a1_ref, z_ref, dinv_ref, o_ref):
        d = dinv_ref[:, 0:1]
        o_ref[...] = (a0_ref[0] + a1_ref[0] + z_ref[...]) * (d * d)

    return pl.pallas_call(
        body,
        grid=(N // _BR,),
        in_specs=[
            pl.BlockSpec((1, _BR, C), lambda i: (0, i, 0)),
            pl.BlockSpec((1, _BR, C), lambda i: (1, i, 0)),
            pl.BlockSpec((_BR, C), lambda i: (i, 0)),
            pl.BlockSpec((_BR, 8), lambda i: (i, 0)),
        ],
        out_specs=pl.BlockSpec((_BR, C), lambda i: (i, 0)),
        out_shape=jax.ShapeDtypeStruct((N, C), jnp.float32),
    )(ap, ap, z0, dinv)


def _tc_final(ap, z1, dinv, b2d):
    def body(a0_ref, a1_ref, z_ref, dinv_ref, b_ref, o_ref):
        d = dinv_ref[:, 0:1]
        t = (a0_ref[0] + a1_ref[0] + z_ref[...]) * d + b_ref[0:1, :]
        m = jnp.max(t, axis=1, keepdims=True)
        e = jnp.exp(t - m)
        s = jnp.sum(e, axis=1, keepdims=True)
        o_ref[...] = t - m - jnp.log(s)

    return pl.pallas_call(
        body,
        grid=(N // _BR,),
        in_specs=[
            pl.BlockSpec((1, _BR, C), lambda i: (0, i, 0)),
            pl.BlockSpec((1, _BR, C), lambda i: (1, i, 0)),
            pl.BlockSpec((_BR, C), lambda i: (i, 0)),
            pl.BlockSpec((_BR, 8), lambda i: (i, 0)),
            pl.BlockSpec((8, C), lambda i: (0, 0)),
        ],
        out_specs=pl.BlockSpec((_BR, C), lambda i: (i, 0)),
        out_shape=jax.ShapeDtypeStruct((N, C), jnp.float32),
    )(ap, ap, z1, dinv, b2d)


def kernel(x, edge_index, W, b):
    src = edge_index[0]
    dst = edge_index[1]
    e = src.shape[0]
    e_pad = _pad_edges(e)
    pad = e_pad - e
    per_w = e_pad // NW
    n_chunks = per_w // CHUNK
    pad_i = jnp.arange(pad, dtype=jnp.int32)
    src_p = jnp.concatenate([src, (pad_i * 37) % N]).reshape(
        NW, n_chunks, CHUNK)
    dst_p = jnp.concatenate([dst, N + pad_i % PAD_ROWS]).reshape(
        NW, n_chunks, CHUNK)
    ones_rows = jnp.ones((CHUNK, DEG_W), jnp.float32)
    b2d = jnp.broadcast_to(b[None, :], (8, C))

    degp = _make_sc_degree(e_pad)(dst_p, ones_rows).reshape(2, NTOT, DEG_W)
    z0, dinv = _tc_head(x, W, degp)
    prop = _make_sc_prop(e_pad)
    a1 = prop(z0, src_p, dst_p).reshape(2, NTOT, C)
    z1 = _tc_mid(a1, z0, dinv)
    a2 = prop(z1, src_p, dst_p).reshape(2, NTOT, C)
    return _tc_final(a2, z1, dinv, b2d)


# speedup vs baseline: 45.0032x; 1.0339x over previous
import functools

import jax
import jax.numpy as jnp
from jax import lax
from jax.experimental import pallas as pl
from jax.experimental.pallas import tpu as pltpu
from jax.experimental.pallas import tpu_sc as plsc

N = 10000
D = 128
C = 40

NC = 2
NS = 16
NW = NC * NS
CHUNK = 128
NBUF = 10
PAD_ROWS = 112
NTOT = N + PAD_ROWS
DEG_W = 8

_SLAB = NTOT // NS
_HSTAGE = 624


def _pad_edges(e):
    block = NW * CHUNK * NBUF
    return ((e + block - 1) // block) * block


def _zero_rows(zbuf, n_rows, width):
    zv = jnp.zeros((16,), jnp.float32)
    cols = [0] if width == 16 else [0, 16, width - 16]

    def body(r, carry):
        for c in cols:
            zbuf[r, pl.ds(c, 16)] = zv
        return carry

    lax.fori_loop(0, n_rows, body, 0)


@functools.lru_cache(maxsize=None)
def _make_sc_degree(e_pad):
    per_w = e_pad // NW
    n_chunks = per_w // CHUNK
    lag = 8
    mesh = plsc.VectorSubcoreMesh(core_axis_name="c", subcore_axis_name="s")

    @functools.partial(
        pl.kernel,
        mesh=mesh,
        out_type=jax.ShapeDtypeStruct((NC * NTOT, DEG_W), jnp.float32),
        compiler_params=pltpu.CompilerParams(use_tc_tiling_on_sc=False),
        scratch_types=[
            pltpu.VMEM((n_chunks, CHUNK), jnp.int32),
            pltpu.VMEM((CHUNK, DEG_W), jnp.float32),
            pltpu.VMEM((CHUNK, DEG_W), jnp.float32),
            pltpu.VMEM_SHARED((NTOT, DEG_W), jnp.float32),
            pltpu.SemaphoreType.DMA,
        ],
    )
    def deg_kernel(didx_hbm, ones_hbm, zeros_hbm, out_hbm, didx_all, ones_v,
                   zbuf, acc, sem):
        cid = lax.axis_index("c")
        sid = lax.axis_index("s")
        wid = sid * NC + cid
        pltpu.sync_copy(zeros_hbm, zbuf)
        base = sid * _SLAB
        for j in range(4):
            pltpu.sync_copy(zbuf, acc.at[pl.ds(base + j * CHUNK, CHUNK)])
        pltpu.sync_copy(zbuf.at[pl.ds(0, _SLAB - 4 * CHUNK)],
                        acc.at[pl.ds(base + 4 * CHUNK, _SLAB - 4 * CHUNK)])
        pltpu.sync_copy(ones_hbm, ones_v)
        pltpu.sync_copy(didx_hbm.at[wid], didx_all)
        plsc.subcore_barrier()
        for i in range(n_chunks):
            pltpu.async_copy(ones_v, acc.at[didx_all.at[i]], sem, add=True)
            if i >= lag:
                pltpu.make_async_copy(
                    ones_v, acc.at[didx_all.at[i - lag]], sem).wait()
        for i in range(n_chunks - lag, n_chunks):
            pltpu.make_async_copy(ones_v, acc.at[didx_all.at[i]], sem).wait()
        plsc.subcore_barrier()
        pltpu.sync_copy(
            acc.at[pl.ds(sid * _SLAB, _SLAB)],
            out_hbm.at[pl.ds(cid * NTOT + sid * _SLAB, _SLAB)],
        )

    return deg_kernel


@functools.lru_cache(maxsize=None)
def _make_sc_prop(e_pad):
    per_w = e_pad // NW
    n_chunks = per_w // CHUNK
    n_groups = n_chunks // NBUF
    mesh = plsc.VectorSubcoreMesh(core_axis_name="c", subcore_axis_name="s")

    @functools.partial(
        pl.kernel,
        mesh=mesh,
        out_type=jax.ShapeDtypeStruct((NC * NTOT, C), jnp.float32),
        compiler_params=pltpu.CompilerParams(use_tc_tiling_on_sc=False),
        scratch_types=[
            pltpu.VMEM((n_chunks, CHUNK), jnp.int32),
            pltpu.VMEM((n_chunks, CHUNK), jnp.int32),
            pltpu.VMEM((NBUF, CHUNK, C), jnp.float32),
            pltpu.VMEM((CHUNK, C), jnp.float32),
            pltpu.VMEM_SHARED((N, C), jnp.float32),
            pltpu.VMEM_SHARED((NTOT, C), jnp.float32),
            pltpu.SemaphoreType.DMA((NBUF,)),
            pltpu.SemaphoreType.DMA((NBUF,)),
        ],
    )
    def prop_kernel(h_hbm, sidx_hbm, didx_hbm, out_hbm,
                    sidx_all, didx_all, rows, zbuf, h_sp, acc, gsem, ssem):
        cid = lax.axis_index("c")
        sid = lax.axis_index("s")
        wid = sid * NC + cid
        pltpu.sync_copy(h_hbm.at[pl.ds(sid * _HSTAGE, _HSTAGE)],
                        h_sp.at[pl.ds(sid * _HSTAGE, _HSTAGE)])

        @pl.when(sid == NS - 1)
        def _():
            pltpu.sync_copy(h_hbm.at[pl.ds(NS * _HSTAGE, N - NS * _HSTAGE)],
                            h_sp.at[pl.ds(NS * _HSTAGE, N - NS * _HSTAGE)])

        _zero_rows(zbuf, CHUNK, C)
        base = sid * _SLAB
        for j in range(4):
            pltpu.sync_copy(zbuf, acc.at[pl.ds(base + j * CHUNK, CHUNK)])
        pltpu.sync_copy(zbuf.at[pl.ds(0, _SLAB - 4 * CHUNK)],
                        acc.at[pl.ds(base + 4 * CHUNK, _SLAB - 4 * CHUNK)])
        pltpu.sync_copy(sidx_hbm.at[wid], sidx_all)
        pltpu.sync_copy(didx_hbm.at[wid], didx_all)
        plsc.subcore_barrier()

        def gather(i, b):
            pltpu.async_copy(h_sp.at[sidx_all.at[i]], rows.at[b], gsem.at[b])

        def gather_wait(i, b):
            pltpu.make_async_copy(
                h_sp.at[sidx_all.at[i]], rows.at[b], gsem.at[b]).wait()

        def scatter(i, b):
            pltpu.async_copy(rows.at[b], acc.at[didx_all.at[i]], ssem.at[b],
                             add=True)

        def scatter_wait(i, b):
            pltpu.make_async_copy(
                rows.at[b], acc.at[didx_all.at[i]], ssem.at[b]).wait()

        for b in range(NBUF):
            gather(b, b)

        def body(g, carry):
            i0 = g * NBUF
            for b in range(NBUF):
                gather_wait(i0 + b, b)
                scatter(i0 + b, b)
            for b in range(NBUF):
                scatter_wait(i0 + b, b)
                gather(i0 + NBUF + b, b)
            return carry

        lax.fori_loop(0, n_groups - 1, body, 0)
        i0 = (n_groups - 1) * NBUF
        for b in range(NBUF):
            gather_wait(i0 + b, b)
            scatter(i0 + b, b)
        for b in range(NBUF):
            scatter_wait(i0 + b, b)
        plsc.subcore_barrier()
        pltpu.sync_copy(
            acc.at[pl.ds(sid * _SLAB, _SLAB)],
            out_hbm.at[pl.ds(cid * NTOT + sid * _SLAB, _SLAB)],
        )

    return prop_kernel


_BR = 2000


def _tc_head(x, W, degp):
    def body(x_ref, w_ref, d0_ref, d1_ref, z_ref, dinv_ref):
        deg = d0_ref[0, :, 0:1] + d1_ref[0, :, 0:1] + 1.0
        dinv = lax.rsqrt(deg)
        y = jnp.dot(x_ref[...], w_ref[...], preferred_element_type=jnp.float32)
        z_ref[...] = y * dinv
        dinv_ref[...] = jnp.broadcast_to(dinv, (_BR, 8))

    return pl.pallas_call(
        body,
        grid=(N // _BR,),
        in_specs=[
            pl.BlockSpec((_BR, D), lambda i: (i, 0)),
            pl.BlockSpec((D, C), lambda i: (0, 0)),
            pl.BlockSpec((1, _BR, DEG_W), lambda i: (0, i, 0)),
            pl.BlockSpec((1, _BR, DEG_W), lambda i: (1, i, 0)),
        ],
        out_specs=[
            pl.BlockSpec((_BR, C), lambda i: (i, 0)),
            pl.BlockSpec((_BR, 8), lambda i: (i, 0)),
        ],
        out_shape=[
            jax.ShapeDtypeStruct((N, C), jnp.float32),
            jax.ShapeDtypeStruct((N, 8), jnp.float32),
        ],
    )(x, W, degp, degp)


def _tc_mid(ap, z0, dinv):
    def body(a0_ref, a1_ref, z_ref, dinv_ref, o_ref):
        d = dinv_ref[:, 0:1]
        o_ref[...] = (a0_ref[0] + a1_ref[0] + z_ref[...]) * (d * d)

    return pl.pallas_call(
        body,
        grid=(N // _BR,),
        in_specs=[
            pl.BlockSpec((1, _BR, C), lambda i: (0, i, 0)),
            pl.BlockSpec((1, _BR, C), lambda i: (1, i, 0)),
            pl.BlockSpec((_BR, C), lambda i: (i, 0)),
            pl.BlockSpec((_BR, 8), lambda i: (i, 0)),
        ],
        out_specs=pl.BlockSpec((_BR, C), lambda i: (i, 0)),
        out_shape=jax.ShapeDtypeStruct((N, C), jnp.float32),
    )(ap, ap, z0, dinv)


def _tc_final(ap, z1, dinv, b2d):
    def body(a0_ref, a1_ref, z_ref, dinv_ref, b_ref, o_ref):
        d = dinv_ref[:, 0:1]
        t = (a0_ref[0] + a1_ref[0] + z_ref[...]) * d + b_ref[0:1, :]
        m = jnp.max(t, axis=1, keepdims=True)
        e = jnp.exp(t - m)
        s = jnp.sum(e, axis=1, keepdims=True)
        o_ref[...] = t - m - jnp.log(s)

    return pl.pallas_call(
        body,
        grid=(N // _BR,),
        in_specs=[
            pl.BlockSpec((1, _BR, C), lambda i: (0, i, 0)),
            pl.BlockSpec((1, _BR, C), lambda i: (1, i, 0)),
            pl.BlockSpec((_BR, C), lambda i: (i, 0)),
            pl.BlockSpec((_BR, 8), lambda i: (i, 0)),
            pl.BlockSpec((8, C), lambda i: (0, 0)),
        ],
        out_specs=pl.BlockSpec((_BR, C), lambda i: (i, 0)),
        out_shape=jax.ShapeDtypeStruct((N, C), jnp.float32),
    )(ap, ap, z1, dinv, b2d)


def kernel(x, edge_index, W, b):
    src = edge_index[0]
    dst = edge_index[1]
    e = src.shape[0]
    e_pad = _pad_edges(e)
    pad = e_pad - e
    per_w = e_pad // NW
    n_chunks = per_w // CHUNK
    pad_i = jnp.arange(pad, dtype=jnp.int32)
    src_p = jnp.concatenate([src, (pad_i * 37) % N]).reshape(
        NW, n_chunks, CHUNK)
    dst_p = jnp.concatenate([dst, N + pad_i % PAD_ROWS]).reshape(
        NW, n_chunks, CHUNK)
    ones_rows = jnp.ones((CHUNK, DEG_W), jnp.float32)
    zeros_rows = jnp.zeros((CHUNK, DEG_W), jnp.float32)
    b2d = jnp.broadcast_to(b[None, :], (8, C))

    degp = _make_sc_degree(e_pad)(dst_p, ones_rows, zeros_rows).reshape(
        2, NTOT, DEG_W)
    z0, dinv = _tc_head(x, W, degp)
    prop = _make_sc_prop(e_pad)
    a1 = prop(z0, src_p, dst_p).reshape(2, NTOT, C)
    z1 = _tc_mid(a1, z0, dinv)
    a2 = prop(z1, src_p, dst_p).reshape(2, NTOT, C)
    return _tc_final(a2, z1, dinv, b2d)
